# baseline (speedup 1.0000x reference)
import functools

import jax
import jax.numpy as jnp
from jax import lax
from jax.experimental import pallas as pl
from jax.experimental.pallas import tpu as pltpu
from jax.experimental.pallas import tpu_sc as plsc

_B = 4096
_L = 200
_D = 64
_NC = 2
_NS = 16
_NW = _NC * _NS
_RPW = _B // _NW
_U = 8
_INV_L = 1.0 / _L


def _pool_body(x_hbm, emb_hbm, z_hbm, idx_v, rows0, rows1, zbuf, sem0, sem1):
    wid = lax.axis_index("s") * _NC + lax.axis_index("c")
    base = wid * _RPW

    pltpu.sync_copy(x_hbm.at[pl.ds(base * _L, _RPW * _L)], idx_v)

    def gather(i, rows_ref, sem):
        off = pl.multiple_of(i * _L, 8)
        pltpu.make_async_copy(
            emb_hbm.at[idx_v.at[pl.ds(off, 128)]],
            rows_ref.at[pl.ds(0, 128)], sem).start()
        pltpu.make_async_copy(
            emb_hbm.at[idx_v.at[pl.ds(off + 128, _L - 128)]],
            rows_ref.at[pl.ds(128, _L - 128)], sem).start()

    def wait_rows(rows_ref, sem):
        pltpu.make_async_copy(emb_hbm.at[pl.ds(0, _L)], rows_ref, sem).wait()

    def reduce_store(rows_ref, i):
        def rbody(jj, accs):
            a0, a1, a2, a3 = accs
            j0 = jj * _U
            for u in range(_U):
                j = j0 + u
                a0 = a0 + rows_ref[j, pl.ds(0, 16)]
                a1 = a1 + rows_ref[j, pl.ds(16, 16)]
                a2 = a2 + rows_ref[j, pl.ds(32, 16)]
                a3 = a3 + rows_ref[j, pl.ds(48, 16)]
            return a0, a1, a2, a3
        zero = jnp.zeros((16,), jnp.float32)
        a0, a1, a2, a3 = lax.fori_loop(0, _L // _U, rbody,
                                       (zero, zero, zero, zero))
        zbuf[i, pl.ds(0, 16)] = a0 * _INV_L
        zbuf[i, pl.ds(16, 16)] = a1 * _INV_L
        zbuf[i, pl.ds(32, 16)] = a2 * _INV_L
        zbuf[i, pl.ds(48, 16)] = a3 * _INV_L

    gather(0, rows0, sem0)

    def body(it, carry):
        i0 = it * 2
        gather(i0 + 1, rows1, sem1)
        wait_rows(rows0, sem0)
        reduce_store(rows0, i0)

        @pl.when(i0 + 2 < _RPW)
        def _():
            gather(i0 + 2, rows0, sem0)

        wait_rows(rows1, sem1)
        reduce_store(rows1, i0 + 1)
        return carry

    lax.fori_loop(0, _RPW // 2, body, 0)

    pltpu.sync_copy(zbuf, z_hbm.at[pl.ds(base, _RPW)])


_pool = functools.partial(
    pl.kernel,
    out_type=jax.ShapeDtypeStruct((_B, _D), jnp.float32),
    mesh=plsc.VectorSubcoreMesh(core_axis_name="c", subcore_axis_name="s"),
    scratch_types=[
        pltpu.VMEM((_RPW * _L,), jnp.int32),
        pltpu.VMEM((_L, _D), jnp.float32),
        pltpu.VMEM((_L, _D), jnp.float32),
        pltpu.VMEM((_RPW, _D), jnp.float32),
        pltpu.SemaphoreType.DMA,
        pltpu.SemaphoreType.DMA,
    ],
    compiler_params=pltpu.CompilerParams(use_tc_tiling_on_sc=False),
)(_pool_body)


def _mlp_body(z_ref, w1_ref, b1_ref, w2_ref, b2_ref, wo_ref, bo_ref, out_ref):
    h = jnp.dot(z_ref[...], w1_ref[...], preferred_element_type=jnp.float32)
    h = jnp.maximum(h + b1_ref[...], 0.0)
    h = jnp.dot(h, w2_ref[...], preferred_element_type=jnp.float32)
    h = jnp.maximum(h + b2_ref[...], 0.0)
    out_ref[...] = (jnp.dot(h, wo_ref[...], preferred_element_type=jnp.float32)
                    + bo_ref[...])


def kernel(x, emb, W1, b1, W2, b2, Wout, bout):
    z = _pool(x.reshape(-1), emb)
    nclass = Wout.shape[1]
    wo_p = jnp.pad(Wout, ((0, 0), (0, 128 - nclass)))
    bo_p = jnp.pad(bout, (0, 128 - nclass)).reshape(1, 128)
    out_p = pl.pallas_call(
        _mlp_body,
        out_shape=jax.ShapeDtypeStruct((_B, 128), jnp.float32),
    )(z, W1, b1.reshape(1, -1), W2, b2.reshape(1, -1), wo_p, bo_p)
    return out_p[:, :nclass]
